# baseline (speedup 1.0000x reference)
import functools

import jax
import jax.numpy as jnp
import numpy as np
from jax import lax
from jax.experimental import pallas as pl
from jax.experimental.pallas import tpu as pltpu
from jax.experimental.pallas import tpu_sc as plsc

_N = 100000
_S = 24576
_K = 16
_B = 8192
_EMBED = 128
_EDGE = 16
_TDIM = 32
_DH = 64

_NC = 2
_NS = 16
_NW = _NC * _NS
_SPW = _S // _NW
_BPW = _B // _NW
_SRW = _S // 128 // _NW
_NRW = _S * _K // 128 // _NW
_GRP = 6
_SB = 256
_NBLK = _S // _SB
_ZPAD = _SB

@functools.cache
def _mesh():
  return plsc.VectorSubcoreMesh(
      core_axis_name="c", subcore_axis_name="s",
      num_cores=_NC, num_subcores=_NS)


def _wid():
  return lax.axis_index("s") * _NC + lax.axis_index("c")


def _take16(x, idx):
  dnums = lax.GatherDimensionNumbers(
      offset_dims=(), collapsed_slice_dims=(0,), start_index_map=(0,))
  return lax.gather(x, idx[:, None], dnums, (1,),
                    mode=lax.GatherScatterMode.PROMISE_IN_BOUNDS)


def _ka_body(g2l_hbm, seeds_hbm, src_hbm, dst_hbm, neg_hbm,
             winners_hbm, gsrc_hbm, gdst_hbm, gneg_hbm,
             idx_v, stage_v, ostage_v):
  wid = _wid()
  lane = lax.iota(jnp.int32, 16)

  def phase1(g2l_v):
    pltpu.sync_copy(g2l_hbm, g2l_v)
    pltpu.sync_copy(seeds_hbm.at[pl.ds(wid * _SPW, _SPW)], stage_v)

    def seed_loop(j, carry):
      sv = stage_v[pl.ds(j * 16, 16)]
      idx_v[pl.ds(j * 16, 16)] = plsc.load_gather(g2l_v, [sv])
      return carry

    lax.fori_loop(0, _SPW // 16, seed_loop, 0)

    for src_h, out_h in ((src_hbm, gsrc_hbm), (dst_hbm, gdst_hbm),
                         (neg_hbm, gneg_hbm)):
      pltpu.sync_copy(src_h.at[pl.ds(wid * _BPW, _BPW)],
                      stage_v.at[pl.ds(0, _BPW)])

      def q_loop(j, carry):
        sv = stage_v[pl.ds(j * 16, 16)]
        ostage_v[pl.ds(j * 16, 16)] = plsc.load_gather(g2l_v, [sv])
        return carry

      lax.fori_loop(0, _BPW // 16, q_loop, 0)
      pltpu.sync_copy(ostage_v, out_h.at[pl.ds(wid * _BPW, _BPW)])

  pl.run_scoped(phase1, pltpu.VMEM((_N,), jnp.int32))

  def phase2(win_v):
    def init_loop(j, carry):
      win_v[pl.ds(j * 16, 16)] = jnp.full((16,), -1, jnp.int32)
      return carry

    lax.fori_loop(0, _S // 16, init_loop, 0)
    perm = jnp.minimum(lane + 1, 15)

    def w_loop(j, carry):
      iv = idx_v[pl.ds(j * 16, 16)]
      key = iv * 16 + lane
      ival = wid * _SPW + j * 16 + lane
      sk, sv = plsc.sort_key_val(key, ival)
      grp = jnp.right_shift(sk, 4)
      nxt = _take16(grp, perm)
      last = jnp.logical_or(grp != nxt, lane == 15)
      plsc.store_scatter(win_v, [grp], sv, mask=last)
      return carry

    lax.fori_loop(0, _SPW // 16, w_loop, 0)
    pltpu.sync_copy(win_v, winners_hbm.at[pl.ds(wid * _S, _S)])

  pl.run_scoped(phase2, pltpu.VMEM((_S,), jnp.int32))


@functools.cache
def _ka():
  return pl.kernel(
      _ka_body,
      out_type=[
          jax.ShapeDtypeStruct((_NW * _S,), jnp.int32),
          jax.ShapeDtypeStruct((_B,), jnp.int32),
          jax.ShapeDtypeStruct((_B,), jnp.int32),
          jax.ShapeDtypeStruct((_B,), jnp.int32),
      ],
      mesh=_mesh(),
      compiler_params=pltpu.CompilerParams(needs_layout_passes=False),
      scratch_types=[
          pltpu.VMEM((_SPW,), jnp.int32),
          pltpu.VMEM((_SPW,), jnp.int32),
          pltpu.VMEM((_BPW,), jnp.int32),
      ])


def _kb_body(winners_hbm, winner_hbm, acc_v, tmp_v):
  wid = _wid()
  base = wid * _SPW
  pltpu.sync_copy(winners_hbm.at[pl.ds(base, _SPW)], acc_v)

  def k_loop(k, carry):
    pltpu.sync_copy(winners_hbm.at[pl.ds(k * _S + base, _SPW)], tmp_v)

    def v_loop(j, c2):
      s = pl.ds(j * 16, 16)
      acc_v[s] = jnp.maximum(acc_v[s], tmp_v[s])
      return c2

    lax.fori_loop(0, _SPW // 16, v_loop, 0)
    return carry

  lax.fori_loop(1, _NW, k_loop, 0)
  pltpu.sync_copy(acc_v, winner_hbm.at[pl.ds(base, _SPW)])


@functools.cache
def _kb():
  return pl.kernel(
      _kb_body,
      out_type=[jax.ShapeDtypeStruct((_S,), jnp.int32)],
      mesh=_mesh(),
      compiler_params=pltpu.CompilerParams(needs_layout_passes=False),
      scratch_types=[
          pltpu.VMEM((_SPW,), jnp.int32),
          pltpu.VMEM((_SPW,), jnp.int32),
      ])


def _kg_body(table_hbm, sidx_hbm, nidx_hbm, nf_hbm, nbrf_hbm,
             sidx_v, nidx_v, buf_v, *sems):
  wid = _wid()
  gsems = sems[:_GRP]
  wsems = sems[_GRP:]
  pltpu.sync_copy(sidx_hbm.at[pl.ds(wid * _SPW, _SPW)], sidx_v)
  pltpu.sync_copy(nidx_hbm.at[pl.ds(wid * _NRW * 128, _NRW * 128)], nidx_v)

  cps = [pltpu.async_copy(table_hbm.at[sidx_v.at[pl.ds(b * 128, 128)]],
                          buf_v.at[b], gsems[b])
         for b in range(_GRP)]
  for b in range(_GRP):
    cps[b].wait()
    pltpu.sync_copy(buf_v.at[b], nf_hbm.at[pl.ds(wid * _SPW + b * 128, 128)])

  def grp_loop(g, carry):
    gcps = [pltpu.async_copy(
        table_hbm.at[nidx_v.at[pl.ds((g * _GRP + b) * 128, 128)]],
        buf_v.at[b], gsems[b])
            for b in range(_GRP)]
    wcps = []
    for b in range(_GRP):
      gcps[b].wait()
      wcps.append(pltpu.async_copy(
          buf_v.at[b],
          nbrf_hbm.at[pl.ds(wid * _NRW * 128 + (g * _GRP + b) * 128, 128)],
          wsems[b]))
    for b in range(_GRP):
      wcps[b].wait()
    return carry

  lax.fori_loop(0, _NRW // _GRP, grp_loop, 0)


@functools.cache
def _kg():
  return pl.kernel(
      _kg_body,
      out_type=[
          jax.ShapeDtypeStruct((_S, _EMBED), jnp.float32),
          jax.ShapeDtypeStruct((_S * _K, _EMBED), jnp.float32),
      ],
      mesh=_mesh(),
      compiler_params=pltpu.CompilerParams(needs_layout_passes=False),
      scratch_types=(
          [pltpu.VMEM((_SPW,), jnp.int32),
           pltpu.VMEM((_NRW * 128,), jnp.int32),
           pltpu.VMEM((_GRP, 128, _EMBED), jnp.float32)]
          + [pltpu.SemaphoreType.DMA] * (2 * _GRP)))


def _kh_body(ndf_ref, nbrf_ref, ef_ref, ntm_ref, tm_ref, msk_ref,
             tw_ref, tb_ref, wq_ref, wk_ref, wv_ref,
             w1_ref, b1_ref, w2_ref, b2_ref, out_ref):
  i = pl.program_id(0)

  @pl.when(i == _NBLK)
  def _():
    out_ref[...] = jnp.zeros_like(out_ref)

  @pl.when(i < _NBLK)
  def _():
    f32 = jnp.float32
    ndf = ndf_ref[...]
    tw = tw_ref[...]
    tb = tb_ref[...]
    wq = wq_ref[...]
    qb = jnp.dot(jnp.cos(tb), wq[128:, :], preferred_element_type=f32)
    q = jnp.dot(ndf, wq[:128, :], preferred_element_type=f32) + qb

    mk2 = msk_ref[...]
    dt2 = (tm_ref[...] - ntm_ref[...]) * mk2
    dt3 = dt2[:, :, None]
    ntf = jnp.cos(dt3 * tw.reshape(1, 1, _TDIM) + tb.reshape(1, 1, _TDIM))
    ntf2 = ntf.reshape(_SB * _K, _TDIM)
    m3 = mk2[:, :, None]

    nbrf = nbrf_ref[...]
    ef = ef_ref[...]
    wk = wk_ref[...]
    wv = wv_ref[...]
    kmat = (jnp.dot(nbrf, wk[:128, :], preferred_element_type=f32)
            + jnp.dot(ef, wk[128:144, :], preferred_element_type=f32)
            + jnp.dot(ntf2, wk[144:, :], preferred_element_type=f32))
    vmat = (jnp.dot(nbrf, wv[:128, :], preferred_element_type=f32)
            + jnp.dot(ef, wv[128:144, :], preferred_element_type=f32)
            + jnp.dot(ntf2, wv[144:, :], preferred_element_type=f32))

    qexp = jnp.broadcast_to(q.reshape(_SB, 1, _EMBED),
                            (_SB, _K, _EMBED)).reshape(_SB * _K, _EMBED)
    prod = qexp * kmat
    scale = 1.0 / np.sqrt(_DH)
    s0 = jnp.sum(prod[:, :_DH], axis=1, keepdims=True) * scale
    s1 = jnp.sum(prod[:, _DH:], axis=1, keepdims=True) * scale

    outs = []
    for s_h, vh in ((s0, vmat[:, :_DH]), (s1, vmat[:, _DH:])):
      sc = s_h.reshape(_SB, _K, 1)
      sc = jnp.where(m3 > 0, sc, -1e9)
      mx = jnp.max(sc, axis=1, keepdims=True)
      e = jnp.exp(sc - mx)
      a = e / jnp.sum(e, axis=1, keepdims=True)
      af = jnp.broadcast_to(a.reshape(_SB * _K, 1), (_SB * _K, _DH))
      outs.append(jnp.sum((af * vh).reshape(_SB, _K, _DH), axis=1))
    attn = jnp.concatenate(outs, axis=1)

    w1 = w1_ref[...]
    h1 = jnp.maximum(
        jnp.dot(attn, w1[:128, :], preferred_element_type=f32)
        + jnp.dot(ndf, w1[128:, :], preferred_element_type=f32)
        + b1_ref[...], 0.0)
    out_ref[...] = (jnp.dot(h1, w2_ref[...], preferred_element_type=f32)
                    + b2_ref[...])


def _kh(node_feat, nbr_feat, ef2, ntm3, tm3, msk3, tw2, tb2,
        wq, wk, wv, w1, b1r, w2, b2r):
  def clamp(i):
    return jnp.minimum(i, _NBLK - 1)

  def full(shape):
    return pl.BlockSpec(shape, lambda i, _s=shape: (0,) * len(_s))

  return pl.pallas_call(
      _kh_body,
      grid=(_NBLK + 1,),
      in_specs=[
          pl.BlockSpec((_SB, _EMBED), lambda i: (clamp(i), 0)),
          pl.BlockSpec((_SB * _K, _EMBED), lambda i: (clamp(i), 0)),
          pl.BlockSpec((_SB * _K, _EDGE), lambda i: (clamp(i), 0)),
          pl.BlockSpec((_SB, _K), lambda i: (clamp(i), 0)),
          pl.BlockSpec((_SB, 1), lambda i: (clamp(i), 0)),
          pl.BlockSpec((_SB, _K), lambda i: (clamp(i), 0)),
          full((1, _TDIM)),
          full((1, _TDIM)),
          full((_EMBED + _TDIM, _EMBED)),
          full((_EMBED + _EDGE + _TDIM, _EMBED)),
          full((_EMBED + _EDGE + _TDIM, _EMBED)),
          full((2 * _EMBED, _EMBED)),
          full((1, _EMBED)),
          full((_EMBED, _EMBED)),
          full((1, _EMBED)),
      ],
      out_specs=pl.BlockSpec((_SB, _EMBED), lambda i: (i, 0)),
      out_shape=jax.ShapeDtypeStruct((_S + _ZPAD, _EMBED), jnp.float32),
  )(node_feat, nbr_feat, ef2, ntm3, tm3, msk3, tw2, tb2,
    wq, wk, wv, w1, b1r, w2, b2r)


def _ke_body(winner_hbm, gsrc_hbm, gdst_hbm, gneg_hbm, outz_hbm,
             zsrc_hbm, zdst_hbm, zneg_hbm,
             win_v, gidx_v, widx_v, rows_v, sem):
  wid = _wid()
  lane = lax.iota(jnp.int32, 16)
  pltpu.sync_copy(winner_hbm, win_v)

  for g_h, z_h in ((gsrc_hbm, zsrc_hbm), (gdst_hbm, zdst_hbm),
                   (gneg_hbm, zneg_hbm)):
    pltpu.sync_copy(g_h.at[pl.ds(wid * _BPW, _BPW)], gidx_v)

    def j_loop(j, carry):
      gv = gidx_v[pl.ds(j * 16, 16)]
      wv = plsc.load_gather(win_v, [gv])
      padbase = (wid * _BPW + j * 16) % _ZPAD
      padv = _S + padbase + lane
      adj = jnp.where(wv < 0, padv, wv)
      widx_v[j // 8, pl.ds((j % 8) * 16, 16)] = adj
      return carry

    lax.fori_loop(0, _BPW // 16, j_loop, 0)
    for h in range(_BPW // 128):
      pltpu.async_copy(outz_hbm.at[widx_v.at[h]],
                       rows_v.at[pl.ds(h * 128, 128)], sem).wait()
    pltpu.sync_copy(rows_v, z_h.at[pl.ds(wid * _BPW, _BPW)])


@functools.cache
def _ke():
  return pl.kernel(
      _ke_body,
      out_type=[
          jax.ShapeDtypeStruct((_B, _EMBED), jnp.float32),
          jax.ShapeDtypeStruct((_B, _EMBED), jnp.float32),
          jax.ShapeDtypeStruct((_B, _EMBED), jnp.float32),
      ],
      mesh=_mesh(),
      compiler_params=pltpu.CompilerParams(needs_layout_passes=False),
      scratch_types=[
          pltpu.VMEM((_S,), jnp.int32),
          pltpu.VMEM((_BPW,), jnp.int32),
          pltpu.VMEM((_BPW // 128, 128), jnp.int32),
          pltpu.VMEM((_BPW, _EMBED), jnp.float32),
          pltpu.SemaphoreType.DMA,
      ])


def _kf_body(zs_ref, zd_ref, zn_ref, f1_ref, f1b_ref, f2_ref, f2b_ref,
             pos_ref, neg_ref):
  f32 = jnp.float32
  f1 = f1_ref[...]
  a = jnp.dot(zs_ref[...], f1[:_EMBED, :], preferred_element_type=f32)
  for z_ref, o_ref in ((zd_ref, pos_ref), (zn_ref, neg_ref)):
    h = jnp.maximum(
        a + jnp.dot(z_ref[...], f1[_EMBED:, :], preferred_element_type=f32)
        + f1b_ref[...], 0.0)
    o = jnp.dot(h, f2_ref[...], preferred_element_type=f32) + f2b_ref[...]
    o_ref[...] = jax.nn.sigmoid(o)


def _kf(zsrc, zdst, zneg, f1, f1b, f2, f2b):
  return pl.pallas_call(
      _kf_body,
      out_shape=[
          jax.ShapeDtypeStruct((_B, 1), jnp.float32),
          jax.ShapeDtypeStruct((_B, 1), jnp.float32),
      ],
  )(zsrc, zdst, zneg, f1, f1b, f2, f2b)


def kernel(static_node_feats, seed_nodes, nbrs, nbr_mask, times, nbr_times,
           nbr_feats, g2l, src, dst, neg, t2v_w, t2v_b, Wq, Wk, Wv, W1, b1,
           W2, b2, fc1_w, fc1_b, fc2_w, fc2_b):
  winners, gsrc, gdst, gneg = _ka()(g2l, seed_nodes, src, dst, neg)
  (winner,) = _kb()(winners)

  node_feat, nbr_feat = _kg()(static_node_feats, seed_nodes,
                              nbrs.reshape(_S * _K))

  outz = _kh(node_feat, nbr_feat,
             nbr_feats.reshape(_S * _K, _EDGE),
             nbr_times,
             times.reshape(_S, 1),
             nbr_mask.astype(jnp.float32),
             t2v_w.reshape(1, _TDIM), t2v_b.reshape(1, _TDIM),
             Wq, Wk, Wv, W1, b1.reshape(1, _EMBED), W2, b2.reshape(1, _EMBED))

  zsrc, zdst, zneg = _ke()(winner, gsrc, gdst, gneg, outz)
  pos, negp = _kf(zsrc, zdst, zneg, fc1_w, fc1_b.reshape(1, _EMBED),
                  fc2_w, fc2_b.reshape(1, 1))
  return pos.reshape(-1), negp.reshape(-1)


# speedup vs baseline: 1.2623x; 1.2623x over previous
import functools

import jax
import jax.numpy as jnp
import numpy as np
from jax import lax
from jax.experimental import pallas as pl
from jax.experimental.pallas import tpu as pltpu
from jax.experimental.pallas import tpu_sc as plsc

_N = 100000
_S = 24576
_K = 16
_B = 8192
_EMBED = 128
_EDGE = 16
_TDIM = 32
_DH = 64

_NC = 2
_NS = 16
_NW = _NC * _NS
_SPW = _S // _NW
_BPW = _B // _NW
_NRW = _S * _K // 128 // _NW
_GRP = 6
_SB = 512
_NBLK = _S // _SB
_ZPAD = _SB
_PB = 1000


@functools.cache
def _mesh():
  return plsc.VectorSubcoreMesh(
      core_axis_name="c", subcore_axis_name="s",
      num_cores=_NC, num_subcores=_NS)


def _wid():
  return lax.axis_index("s") * _NC + lax.axis_index("c")


def _take16(x, idx):
  dnums = lax.GatherDimensionNumbers(
      offset_dims=(), collapsed_slice_dims=(0,), start_index_map=(0,))
  return lax.gather(x, idx[:, None], dnums, (1,),
                    mode=lax.GatherScatterMode.PROMISE_IN_BOUNDS)


def _kp_body(snf_ref, wk_ref, wv_ref, out_ref):
  bf = jnp.bfloat16
  f32 = jnp.float32
  x = snf_ref[...].astype(bf)
  pk = jnp.dot(x, wk_ref[...].astype(bf), preferred_element_type=f32)
  pv = jnp.dot(x, wv_ref[...].astype(bf), preferred_element_type=f32)
  k16 = lax.bitcast_convert_type(pk.astype(bf), jnp.int16)
  v16 = lax.bitcast_convert_type(pv.astype(bf), jnp.int16)
  k32 = jnp.bitwise_and(k16.astype(jnp.int32), jnp.int32(0xFFFF))
  v32 = jnp.left_shift(v16.astype(jnp.int32), 16)
  out_ref[...] = jnp.bitwise_or(k32, v32)


def _kp(snf, wk128, wv128):
  return pl.pallas_call(
      _kp_body,
      grid=(_N // _PB,),
      in_specs=[
          pl.BlockSpec((_PB, _EMBED), lambda i: (i, 0)),
          pl.BlockSpec((_EMBED, _EMBED), lambda i: (0, 0)),
          pl.BlockSpec((_EMBED, _EMBED), lambda i: (0, 0)),
      ],
      out_specs=pl.BlockSpec((_PB, _EMBED), lambda i: (i, 0)),
      out_shape=jax.ShapeDtypeStruct((_N, _EMBED), jnp.int32),
  )(snf, wk128, wv128)


def _ka_body(g2l_hbm, seeds_hbm, src_hbm, dst_hbm, neg_hbm,
             winners_hbm, gsrc_hbm, gdst_hbm, gneg_hbm,
             idx_v, stage_v, ostage_v):
  wid = _wid()
  lane = lax.iota(jnp.int32, 16)

  def phase1(g2l_v):
    pltpu.sync_copy(g2l_hbm, g2l_v)
    pltpu.sync_copy(seeds_hbm.at[pl.ds(wid * _SPW, _SPW)], stage_v)

    def seed_loop(j, carry):
      sv = stage_v[pl.ds(j * 16, 16)]
      idx_v[pl.ds(j * 16, 16)] = plsc.load_gather(g2l_v, [sv])
      return carry

    lax.fori_loop(0, _SPW // 16, seed_loop, 0)

    for src_h, out_h in ((src_hbm, gsrc_hbm), (dst_hbm, gdst_hbm),
                         (neg_hbm, gneg_hbm)):
      pltpu.sync_copy(src_h.at[pl.ds(wid * _BPW, _BPW)],
                      stage_v.at[pl.ds(0, _BPW)])

      def q_loop(j, carry):
        sv = stage_v[pl.ds(j * 16, 16)]
        ostage_v[pl.ds(j * 16, 16)] = plsc.load_gather(g2l_v, [sv])
        return carry

      lax.fori_loop(0, _BPW // 16, q_loop, 0)
      pltpu.sync_copy(ostage_v, out_h.at[pl.ds(wid * _BPW, _BPW)])

  pl.run_scoped(phase1, pltpu.VMEM((_N,), jnp.int32))

  def phase2(win_v):
    def init_loop(j, carry):
      win_v[pl.ds(j * 16, 16)] = jnp.full((16,), -1, jnp.int32)
      return carry

    lax.fori_loop(0, _S // 16, init_loop, 0)
    perm = jnp.minimum(lane + 1, 15)

    def w_loop(j, carry):
      iv = idx_v[pl.ds(j * 16, 16)]
      key = iv * 16 + lane
      ival = wid * _SPW + j * 16 + lane
      sk, sv = plsc.sort_key_val(key, ival)
      grp = jnp.right_shift(sk, 4)
      nxt = _take16(grp, perm)
      last = jnp.logical_or(grp != nxt, lane == 15)
      plsc.store_scatter(win_v, [grp], sv, mask=last)
      return carry

    lax.fori_loop(0, _SPW // 16, w_loop, 0)
    pltpu.sync_copy(win_v, winners_hbm.at[pl.ds(wid * _S, _S)])

  pl.run_scoped(phase2, pltpu.VMEM((_S,), jnp.int32))


@functools.cache
def _ka():
  return pl.kernel(
      _ka_body,
      out_type=[
          jax.ShapeDtypeStruct((_NW * _S,), jnp.int32),
          jax.ShapeDtypeStruct((_B,), jnp.int32),
          jax.ShapeDtypeStruct((_B,), jnp.int32),
          jax.ShapeDtypeStruct((_B,), jnp.int32),
      ],
      mesh=_mesh(),
      compiler_params=pltpu.CompilerParams(needs_layout_passes=False),
      scratch_types=[
          pltpu.VMEM((_SPW,), jnp.int32),
          pltpu.VMEM((_SPW,), jnp.int32),
          pltpu.VMEM((_BPW,), jnp.int32),
      ])


def _kb_body(winners_hbm, winner_hbm, acc_v, tmp_v):
  wid = _wid()
  base = wid * _SPW
  pltpu.sync_copy(winners_hbm.at[pl.ds(base, _SPW)], acc_v)

  def k_loop(k, carry):
    pltpu.sync_copy(winners_hbm.at[pl.ds(k * _S + base, _SPW)], tmp_v)

    def v_loop(j, c2):
      s = pl.ds(j * 16, 16)
      acc_v[s] = jnp.maximum(acc_v[s], tmp_v[s])
      return c2

    lax.fori_loop(0, _SPW // 16, v_loop, 0)
    return carry

  lax.fori_loop(1, _NW, k_loop, 0)
  pltpu.sync_copy(acc_v, winner_hbm.at[pl.ds(base, _SPW)])


@functools.cache
def _kb():
  return pl.kernel(
      _kb_body,
      out_type=[jax.ShapeDtypeStruct((_S,), jnp.int32)],
      mesh=_mesh(),
      compiler_params=pltpu.CompilerParams(needs_layout_passes=False),
      scratch_types=[
          pltpu.VMEM((_SPW,), jnp.int32),
          pltpu.VMEM((_SPW,), jnp.int32),
      ])


def _kg_body(table_hbm, pkv_hbm, sidx_hbm, nidx_hbm, nf_hbm, kv_hbm,
             sidx_v, nidx_v, *sems):
  wid = _wid()
  gsems = sems[:_GRP]
  wsems = sems[_GRP:]
  pltpu.sync_copy(sidx_hbm.at[pl.ds(wid * _SPW, _SPW)], sidx_v)
  pltpu.sync_copy(nidx_hbm.at[pl.ds(wid * _NRW * 128, _NRW * 128)], nidx_v)

  def seeds(fbuf_v):
    cps = [None, None]
    for b in range(2):
      cps[b] = pltpu.async_copy(
          table_hbm.at[sidx_v.at[pl.ds(b * 128, 128)]], fbuf_v.at[b],
          gsems[b])
    for c in range(_SPW // 128):
      cps[c % 2].wait()
      pltpu.sync_copy(fbuf_v.at[c % 2],
                      nf_hbm.at[pl.ds(wid * _SPW + c * 128, 128)])
      if c + 2 < _SPW // 128:
        cps[c % 2] = pltpu.async_copy(
            table_hbm.at[sidx_v.at[pl.ds((c + 2) * 128, 128)]],
            fbuf_v.at[c % 2], gsems[c % 2])

  pl.run_scoped(seeds, pltpu.VMEM((2, 128, _EMBED), jnp.float32))

  def nbrs(ibuf_v):
    def grp_loop(g, carry):
      gcps = [pltpu.async_copy(
          pkv_hbm.at[nidx_v.at[pl.ds((g * _GRP + b) * 128, 128)]],
          ibuf_v.at[b], gsems[b])
              for b in range(_GRP)]
      wcps = []
      for b in range(_GRP):
        gcps[b].wait()
        wcps.append(pltpu.async_copy(
            ibuf_v.at[b],
            kv_hbm.at[pl.ds(wid * _NRW * 128 + (g * _GRP + b) * 128, 128)],
            wsems[b]))
      for b in range(_GRP):
        wcps[b].wait()
      return carry

    lax.fori_loop(0, _NRW // _GRP, grp_loop, 0)

  pl.run_scoped(nbrs, pltpu.VMEM((_GRP, 128, _EMBED), jnp.int32))


@functools.cache
def _kg():
  return pl.kernel(
      _kg_body,
      out_type=[
          jax.ShapeDtypeStruct((_S, _EMBED), jnp.float32),
          jax.ShapeDtypeStruct((_S * _K, _EMBED), jnp.int32),
      ],
      mesh=_mesh(),
      compiler_params=pltpu.CompilerParams(needs_layout_passes=False),
      scratch_types=(
          [pltpu.VMEM((_SPW,), jnp.int32),
           pltpu.VMEM((_NRW * 128,), jnp.int32)]
          + [pltpu.SemaphoreType.DMA] * (2 * _GRP)))


def _kh_body(ndf_ref, kv_ref, ef_ref, ntm_ref, tm_ref, msk_ref,
             tb_ref, exp_ref, btile_ref, wq_ref, wke_ref, wve_ref,
             wkte_ref, seg_ref, atile_ref, segwv_ref,
             w1_ref, b1_ref, w2_ref, b2_ref, out_ref):
  i = pl.program_id(0)

  @pl.when(i == _NBLK)
  def _():
    out_ref[...] = jnp.zeros_like(out_ref)

  @pl.when(i < _NBLK)
  def _():
    f32 = jnp.float32
    bf = jnp.bfloat16
    ndf = ndf_ref[...]
    tb = tb_ref[...]
    wq = wq_ref[...]
    qb = jnp.dot(jnp.cos(tb), wq[128:, :], preferred_element_type=f32)
    q = jnp.dot(ndf, wq[:128, :], preferred_element_type=f32) + qb

    mk2 = msk_ref[...]
    dt2 = (tm_ref[...] - ntm_ref[...]) * mk2
    m3 = mk2[:, :, None]

    ang = jnp.dot(dt2, exp_ref[...], preferred_element_type=f32)
    nt = jnp.cos(ang + btile_ref[...])

    w32 = kv_ref[...]
    kp = lax.bitcast_convert_type(jnp.left_shift(w32, 16), f32)
    vp = lax.bitcast_convert_type(
        jnp.bitwise_and(w32, jnp.int32(-65536)), f32)
    ef = ef_ref[...].astype(bf)
    wke = wke_ref[...]
    wve = wve_ref[...]
    kmat = kp + jnp.dot(ef, wke.astype(bf), preferred_element_type=f32)
    vmat = vp + jnp.dot(ef, wve.astype(bf), preferred_element_type=f32)

    qexp = jnp.broadcast_to(q.reshape(_SB, 1, _EMBED),
                            (_SB, _K, _EMBED)).reshape(_SB * _K, _EMBED)
    prod = qexp * kmat
    scale = 1.0 / np.sqrt(_DH)
    s0 = jnp.sum(prod[:, :_DH], axis=1, keepdims=True) * scale
    s1 = jnp.sum(prod[:, _DH:], axis=1, keepdims=True) * scale

    wkte = wkte_ref[...]
    seg = seg_ref[...]
    atile = atile_ref[...]
    segwv = segwv_ref[...]
    outs = []
    for h, (s_h, vh) in enumerate(((s0, vmat[:, :_DH]),
                                   (s1, vmat[:, _DH:]))):
      hs = slice(h * _DH, (h + 1) * _DH)
      qktt = jnp.dot(q[:, hs], wkte[hs, :], preferred_element_type=f32)
      st = jnp.dot(nt * qktt, seg, preferred_element_type=f32)
      sc = s_h.reshape(_SB, _K, 1) + st[:, :, None] * scale
      sc = jnp.where(m3 > 0, sc, -1e9)
      mx = jnp.max(sc, axis=1, keepdims=True)
      e = jnp.exp(sc - mx)
      a = e / jnp.sum(e, axis=1, keepdims=True)
      af = jnp.broadcast_to(a.reshape(_SB * _K, 1), (_SB * _K, _DH))
      avp = jnp.sum((af * vh).reshape(_SB, _K, _DH), axis=1)
      al = a.reshape(_SB, _K)
      ati = jnp.dot(al, atile, preferred_element_type=f32)
      avt = jnp.dot(nt * ati, segwv[:, hs], preferred_element_type=f32)
      outs.append(avp + avt)
    attn = jnp.concatenate(outs, axis=1)

    w1 = w1_ref[...]
    h1 = jnp.maximum(
        jnp.dot(attn, w1[:128, :], preferred_element_type=f32)
        + jnp.dot(ndf, w1[128:, :], preferred_element_type=f32)
        + b1_ref[...], 0.0)
    out_ref[...] = (jnp.dot(h1, w2_ref[...], preferred_element_type=f32)
                    + b2_ref[...])


def _kh(node_feat, kv, ef2, ntm2, tm2, msk2, tb2, exp16, btile,
        wq, wke, wve, wkte, seg, atile, segwv, w1, b1r, w2, b2r):
  def clamp(i):
    return jnp.minimum(i, _NBLK - 1)

  def full(shape):
    return pl.BlockSpec(shape, lambda i, _s=shape: (0,) * len(_s))

  return pl.pallas_call(
      _kh_body,
      grid=(_NBLK + 1,),
      in_specs=[
          pl.BlockSpec((_SB, _EMBED), lambda i: (clamp(i), 0)),
          pl.BlockSpec((_SB * _K, _EMBED), lambda i: (clamp(i), 0)),
          pl.BlockSpec((_SB * _K, _EDGE), lambda i: (clamp(i), 0)),
          pl.BlockSpec((_SB, _K), lambda i: (clamp(i), 0)),
          pl.BlockSpec((_SB, 1), lambda i: (clamp(i), 0)),
          pl.BlockSpec((_SB, _K), lambda i: (clamp(i), 0)),
          full((1, _TDIM)),
          full((_K, _K * _TDIM)),
          full((1, _K * _TDIM)),
          full((_EMBED + _TDIM, _EMBED)),
          full((_EDGE, _EMBED)),
          full((_EDGE, _EMBED)),
          full((_EMBED, _K * _TDIM)),
          full((_K * _TDIM, _K)),
          full((_K, _K * _TDIM)),
          full((_K * _TDIM, _EMBED)),
          full((2 * _EMBED, _EMBED)),
          full((1, _EMBED)),
          full((_EMBED, _EMBED)),
          full((1, _EMBED)),
      ],
      out_specs=pl.BlockSpec((_SB, _EMBED), lambda i: (i, 0)),
      out_shape=jax.ShapeDtypeStruct((_S + _ZPAD, _EMBED), jnp.float32),
  )(node_feat, kv, ef2, ntm2, tm2, msk2, tb2, exp16, btile,
    wq, wke, wve, wkte, seg, atile, segwv, w1, b1r, w2, b2r)


def _ke_body(winner_hbm, gsrc_hbm, gdst_hbm, gneg_hbm, outz_hbm,
             zsrc_hbm, zdst_hbm, zneg_hbm,
             win_v, gidx_v, widx_v, rows_v, sem):
  wid = _wid()
  lane = lax.iota(jnp.int32, 16)
  pltpu.sync_copy(winner_hbm, win_v)

  for g_h, z_h in ((gsrc_hbm, zsrc_hbm), (gdst_hbm, zdst_hbm),
                   (gneg_hbm, zneg_hbm)):
    pltpu.sync_copy(g_h.at[pl.ds(wid * _BPW, _BPW)], gidx_v)

    def j_loop(j, carry):
      gv = gidx_v[pl.ds(j * 16, 16)]
      wv = plsc.load_gather(win_v, [gv])
      padbase = (wid * _BPW + j * 16) % _ZPAD
      padv = _S + padbase + lane
      adj = jnp.where(wv < 0, padv, wv)
      widx_v[j // 8, pl.ds((j % 8) * 16, 16)] = adj
      return carry

    lax.fori_loop(0, _BPW // 16, j_loop, 0)
    for h in range(_BPW // 128):
      pltpu.async_copy(outz_hbm.at[widx_v.at[h]],
                       rows_v.at[pl.ds(h * 128, 128)], sem).wait()
    pltpu.sync_copy(rows_v, z_h.at[pl.ds(wid * _BPW, _BPW)])


@functools.cache
def _ke():
  return pl.kernel(
      _ke_body,
      out_type=[
          jax.ShapeDtypeStruct((_B, _EMBED), jnp.float32),
          jax.ShapeDtypeStruct((_B, _EMBED), jnp.float32),
          jax.ShapeDtypeStruct((_B, _EMBED), jnp.float32),
      ],
      mesh=_mesh(),
      compiler_params=pltpu.CompilerParams(needs_layout_passes=False),
      scratch_types=[
          pltpu.VMEM((_S,), jnp.int32),
          pltpu.VMEM((_BPW,), jnp.int32),
          pltpu.VMEM((_BPW // 128, 128), jnp.int32),
          pltpu.VMEM((_BPW, _EMBED), jnp.float32),
          pltpu.SemaphoreType.DMA,
      ])


def _kf_body(zs_ref, zd_ref, zn_ref, f1_ref, f1b_ref, f2_ref, f2b_ref,
             pos_ref, neg_ref):
  f32 = jnp.float32
  f1 = f1_ref[...]
  a = jnp.dot(zs_ref[...], f1[:_EMBED, :], preferred_element_type=f32)
  for z_ref, o_ref in ((zd_ref, pos_ref), (zn_ref, neg_ref)):
    h = jnp.maximum(
        a + jnp.dot(z_ref[...], f1[_EMBED:, :], preferred_element_type=f32)
        + f1b_ref[...], 0.0)
    o = jnp.dot(h, f2_ref[...], preferred_element_type=f32) + f2b_ref[...]
    o_ref[...] = jax.nn.sigmoid(o)


def _kf(zsrc, zdst, zneg, f1, f1b, f2, f2b):
  return pl.pallas_call(
      _kf_body,
      out_shape=[
          jax.ShapeDtypeStruct((_B, 1), jnp.float32),
          jax.ShapeDtypeStruct((_B, 1), jnp.float32),
      ],
  )(zsrc, zdst, zneg, f1, f1b, f2, f2b)


def kernel(static_node_feats, seed_nodes, nbrs, nbr_mask, times, nbr_times,
           nbr_feats, g2l, src, dst, neg, t2v_w, t2v_b, Wq, Wk, Wv, W1, b1,
           W2, b2, fc1_w, fc1_b, fc2_w, fc2_b):
  pkv = _kp(static_node_feats, Wk[:_EMBED], Wv[:_EMBED])

  winners, gsrc, gdst, gneg = _ka()(g2l, seed_nodes, src, dst, neg)
  (winner,) = _kb()(winners)

  node_feat, kv = _kg()(static_node_feats, pkv, seed_nodes,
                        nbrs.reshape(_S * _K))

  eye16 = jnp.eye(_K, dtype=jnp.float32)
  exp16 = jnp.kron(eye16, t2v_w.reshape(1, _TDIM))
  btile = jnp.tile(t2v_b.reshape(1, _TDIM), (1, _K))
  wkte = jnp.tile(Wk[_EMBED + _EDGE:].T, (1, _K))
  seg = jnp.kron(eye16, jnp.ones((_TDIM, 1), jnp.float32))
  atile = jnp.kron(eye16, jnp.ones((1, _TDIM), jnp.float32))
  segwv = jnp.tile(Wv[_EMBED + _EDGE:], (_K, 1))
  outz = _kh(node_feat, kv,
             nbr_feats.reshape(_S * _K, _EDGE),
             nbr_times,
             times.reshape(_S, 1),
             nbr_mask.astype(jnp.float32),
             t2v_b.reshape(1, _TDIM), exp16, btile,
             Wq, Wk[_EMBED:_EMBED + _EDGE], Wv[_EMBED:_EMBED + _EDGE],
             wkte, seg, atile, segwv,
             W1, b1.reshape(1, _EMBED), W2, b2.reshape(1, _EMBED))

  zsrc, zdst, zneg = _ke()(winner, gsrc, gdst, gneg, outz)
  pos, negp = _kf(zsrc, zdst, zneg, fc1_w, fc1_b.reshape(1, _EMBED),
                  fc2_w, fc2_b.reshape(1, 1))
  return pos.reshape(-1), negp.reshape(-1)
